# baseline (speedup 1.0000x reference)
import functools

import jax
import jax.numpy as jnp
from jax import lax
from jax.experimental import pallas as pl
from jax.experimental.pallas import tpu as pltpu
from jax.experimental.pallas import tpu_sc as plsc


def _proj_body(t_ref, w_ref, o_ref):
    o_ref[...] = jnp.dot(
        t_ref[...], w_ref[...], preferred_element_type=jnp.float32,
    )


def _project(table8, W512, rb=1000):
    V8, D8 = table8.shape
    R8 = W512.shape[1]
    return pl.pallas_call(
        _proj_body,
        grid=(V8 // rb,),
        in_specs=[
            pl.BlockSpec((rb, D8), lambda i: (i, 0)),
            pl.BlockSpec((D8, R8), lambda i: (0, 0)),
        ],
        out_specs=pl.BlockSpec((rb, R8), lambda i: (i, 0)),
        out_shape=jax.ShapeDtypeStruct((V8, R8), jnp.float32),
    )(table8, W512)


def _sc_gather_sum(tokens, P):
    L, B = tokens.shape
    R = P.shape[1]
    info = plsc.get_sparse_core_info()
    NC, NS = info.num_cores, info.num_subcores
    NW = NC * NS
    BPW = B // NW
    CB = 128
    NCH = BPW // CB
    G = 10
    NG = L // G
    GC = 128 // R
    NRG = CB // GC
    mesh = plsc.VectorSubcoreMesh(core_axis_name="c", subcore_axis_name="s")

    @functools.partial(
        pl.kernel,
        mesh=mesh,
        compiler_params=pltpu.CompilerParams(use_tc_tiling_on_sc=False),
        out_type=jax.ShapeDtypeStruct((B, R), jnp.float32),
        scratch_types=[
            pltpu.VMEM((L, CB), jnp.int32),
            pltpu.VMEM((L * CB, R), jnp.float32),
            pltpu.VMEM((CB, R), jnp.float32),
            pltpu.SemaphoreType.DMA,
        ],
    )
    def k(tokens_hbm, p_hbm, out_hbm, idx_v, rows_v, outst_v, sem):
        wid = lax.axis_index("s") * NC + lax.axis_index("c")
        base = wid * BPW

        for ch in range(NCH):
            cbase = base + ch * CB
            pltpu.sync_copy(tokens_hbm.at[:, pl.ds(cbase, CB)], idx_v)

            def gather_group(g, _):
                hs = []
                for j in range(G):
                    l = g * G + j
                    hs.append(pltpu.async_copy(
                        p_hbm.at[idx_v.at[l]],
                        rows_v.at[pl.ds(l * CB, CB)],
                        sem,
                    ))
                for h in hs:
                    h.wait()
                return 0

            lax.fori_loop(0, NG, gather_group, 0)

            def red_group(rg, _):
                col0 = rg * GC

                def l_body(l, accs):
                    rbase = l * CB + col0
                    return tuple(
                        tuple(accs[c][v] + rows_v[rbase + c, pl.ds(v * 16, 16)]
                              for v in range(R // 16))
                        for c in range(GC)
                    )

                zero = jnp.zeros((16,), jnp.float32)
                init = tuple(tuple(zero for _ in range(R // 16))
                             for _ in range(GC))
                accs = lax.fori_loop(0, L, l_body, init)
                for c in range(GC):
                    for v in range(R // 16):
                        outst_v[col0 + c, pl.ds(v * 16, 16)] = accs[c][v]
                return 0

            lax.fori_loop(0, NRG, red_group, 0)
            pltpu.sync_copy(outst_v, out_hbm.at[pl.ds(cbase, CB)])

    return k(tokens, P)


def _head_body(L, C, s_ref, len_ref, p_ref, o_ref):
    lf = len_ref[...]
    inv_l = jnp.float32(1.0 / L)
    cols = []
    for c in range(C):
        cols.append(s_ref[:, c:c + 1] * inv_l
                    + lf * p_ref[0, c] + p_ref[1, c])
    o = jnp.concatenate(cols, axis=1)
    m = jnp.max(o, axis=1, keepdims=True)
    lse = m + jnp.log(jnp.sum(jnp.exp(o - m), axis=1, keepdims=True))
    o_ref[...] = o - lse


def _head(S, lengths_f, params, L, bb=2048):
    B, R = S.shape
    C = params.shape[1]
    return pl.pallas_call(
        functools.partial(_head_body, L, C),
        grid=(B // bb,),
        in_specs=[
            pl.BlockSpec((bb, R), lambda i: (i, 0)),
            pl.BlockSpec((bb, 1), lambda i: (i, 0)),
            pl.BlockSpec(memory_space=pltpu.SMEM),
        ],
        out_specs=pl.BlockSpec((bb, C), lambda i: (i, 0)),
        out_shape=jax.ShapeDtypeStruct((B, C), jnp.float32),
    )(S, lengths_f, params)


def kernel(tokens, lengths, table, W, b):
    L, B = tokens.shape
    V, D = table.shape
    C = W.shape[0]
    R = 16

    Wp = jnp.zeros((D, R), jnp.float32).at[:, :C].set(W[:, :D].T)
    W512 = jnp.kron(jnp.eye(8, dtype=jnp.float32), Wp)
    table8 = table.reshape(V // 8, 8 * D)
    P8 = _project(table8, W512)
    P = P8.reshape(V, R)
    S = _sc_gather_sum(tokens, P)

    params = jnp.stack([W[:, D], b])
    lengths_f = lengths.astype(jnp.float32).reshape(B, 1)
    return _head(S, lengths_f, params, L)
